# baseline (speedup 1.0000x reference)
import jax
import jax.numpy as jnp
from jax.experimental import pallas as pl
from jax.experimental.pallas import tpu as pltpu

N = 4096
D = 512
BM = 256
NB = N // BM


def _pair_body(emb_ref, w1t_ref, w2t_ref, bfc1_ref, bfc2_ref, bias1_ref,
               bias2_ref, p1_ref, p2_ref, wslat_ref, bsla_ref,
               meta1_ref, meta2_ref,
               v1_ref, v2_ref, acc1_ref, acc2_ref, h1_scr, h2_scr,
               a1_scr, a2_scr):
    i = pl.program_id(0)

    @pl.when(i == 0)
    def _init():
        h1 = jnp.dot(emb_ref[...], w1t_ref[...],
                     preferred_element_type=jnp.float32) + bfc1_ref[...]
        h1_scr[...] = h1.astype(jnp.bfloat16)
        h2 = jnp.dot(emb_ref[...], w2t_ref[...],
                     preferred_element_type=jnp.float32) + bfc2_ref[...]
        h2_scr[...] = h2.astype(jnp.bfloat16)
        a1_scr[...] = jnp.zeros_like(a1_scr)
        a2_scr[...] = jnp.zeros_like(a2_scr)

    out1 = jnp.dot(meta1_ref[...].astype(jnp.bfloat16), h1_scr[...],
                   preferred_element_type=jnp.float32) + bias1_ref[...]
    vb1 = jnp.where(out1 >= 0, out1, p1_ref[0, 0] * out1).astype(jnp.bfloat16)
    v1_ref[...] = vb1
    out2 = jnp.dot(meta2_ref[...].astype(jnp.bfloat16), h2_scr[...],
                   preferred_element_type=jnp.float32) + bias2_ref[...]
    vb2 = jnp.where(out2 >= 0, out2, p2_ref[0, 0] * out2).astype(jnp.bfloat16)
    v2_ref[...] = vb2

    s1 = jnp.tanh(jnp.dot(vb1, wslat_ref[...],
                          preferred_element_type=jnp.float32) + bsla_ref[...])
    a1_scr[...] += jnp.sum(s1, axis=0, keepdims=True)
    s2 = jnp.tanh(jnp.dot(vb2, wslat_ref[...],
                          preferred_element_type=jnp.float32) + bsla_ref[...])
    a2_scr[...] += jnp.sum(s2, axis=0, keepdims=True)

    @pl.when(i == NB - 1)
    def _fin():
        acc1_ref[...] = a1_scr[...]
        acc2_ref[...] = a2_scr[...]


def _pair_call(emb_bf, w1t, w2t, bfc1, bfc2, bias1, bias2, p1, p2,
               wslat, bsla, meta1, meta2):
    const = lambda i: (0, 0)
    mblk = pl.BlockSpec((BM, N), lambda i: (i, 0))
    vblk = pl.BlockSpec((BM, D), lambda i: (i, 0))
    return pl.pallas_call(
        _pair_body,
        grid=(NB,),
        in_specs=[
            pl.BlockSpec((N, D), const),
            pl.BlockSpec((D, D), const),
            pl.BlockSpec((D, D), const),
            pl.BlockSpec((1, D), const),
            pl.BlockSpec((1, D), const),
            pl.BlockSpec((1, D), const),
            pl.BlockSpec((1, D), const),
            pl.BlockSpec((1, 1), const),
            pl.BlockSpec((1, 1), const),
            pl.BlockSpec((D, D), const),
            pl.BlockSpec((1, D), const),
            mblk,
            mblk,
        ],
        out_specs=[
            vblk,
            vblk,
            pl.BlockSpec((1, D), const),
            pl.BlockSpec((1, D), const),
        ],
        out_shape=[
            jax.ShapeDtypeStruct((N, D), jnp.bfloat16),
            jax.ShapeDtypeStruct((N, D), jnp.bfloat16),
            jax.ShapeDtypeStruct((1, D), jnp.float32),
            jax.ShapeDtypeStruct((1, D), jnp.float32),
        ],
        scratch_shapes=[
            pltpu.VMEM((N, D), jnp.bfloat16),
            pltpu.VMEM((N, D), jnp.bfloat16),
            pltpu.VMEM((1, D), jnp.float32),
            pltpu.VMEM((1, D), jnp.float32),
        ],
    )(emb_bf, w1t, w2t, bfc1, bfc2, bias1, bias2, p1, p2, wslat, bsla,
      meta1, meta2)


def _combine_body(acc1_ref, acc2_ref, asla_ref, v1_ref, v2_ref, o_ref):
    la = jnp.sum(asla_ref[...] * acc1_ref[...] * (1.0 / N),
                 axis=1, keepdims=True)
    lb = jnp.sum(asla_ref[...] * acc2_ref[...] * (1.0 / N),
                 axis=1, keepdims=True)
    m = jnp.maximum(la, lb)
    ea = jnp.exp(la - m)
    eb = jnp.exp(lb - m)
    inv = 1.0 / (ea + eb)
    o_ref[...] = (v1_ref[...].astype(jnp.float32) * (ea * inv)
                  + v2_ref[...].astype(jnp.float32) * (eb * inv))


def _combine_call(acc1, acc2, asla, v1, v2):
    const = lambda i: (0, 0)
    vblk = pl.BlockSpec((BM, D), lambda i: (i, 0))
    return pl.pallas_call(
        _combine_body,
        grid=(NB,),
        in_specs=[
            pl.BlockSpec((1, D), const),
            pl.BlockSpec((1, D), const),
            pl.BlockSpec((1, D), const),
            vblk,
            vblk,
        ],
        out_specs=vblk,
        out_shape=jax.ShapeDtypeStruct((N, D), jnp.float32),
    )(acc1, acc2, asla, v1, v2)


@jax.jit
def kernel(emb_mi, emb_di, meta_mdm, meta_mdmdm, meta_dmd, meta_dmdmd,
           W_mdm, bfc_mdm, bias_mdm, p_mdm,
           W_mdmdm, bfc_mdmdm, bias_mdmdm, p_mdmdm,
           W_dmd, bfc_dmd, bias_dmd, p_dmd,
           W_dmdmd, bfc_dmdmd, bias_dmdmd, p_dmdmd,
           W_sla, b_sla, a_sla):
    wslat = W_sla.T.astype(jnp.bfloat16)
    bsla = b_sla.reshape(1, D)
    asla = a_sla.reshape(1, D)

    v1, v2, acc1, acc2 = _pair_call(
        emb_mi.astype(jnp.bfloat16),
        W_mdm.T.astype(jnp.bfloat16), W_mdmdm.T.astype(jnp.bfloat16),
        bfc_mdm.reshape(1, D), bfc_mdmdm.reshape(1, D),
        bias_mdm.reshape(1, D), bias_mdmdm.reshape(1, D),
        p_mdm.reshape(1, 1), p_mdmdm.reshape(1, 1),
        wslat, bsla, meta_mdm, meta_mdmdm)
    v3, v4, acc3, acc4 = _pair_call(
        emb_di.astype(jnp.bfloat16),
        W_dmd.T.astype(jnp.bfloat16), W_dmdmd.T.astype(jnp.bfloat16),
        bfc_dmd.reshape(1, D), bfc_dmdmd.reshape(1, D),
        bias_dmd.reshape(1, D), bias_dmdmd.reshape(1, D),
        p_dmd.reshape(1, 1), p_dmdmd.reshape(1, 1),
        wslat, bsla, meta_dmd, meta_dmdmd)
    out_mi = _combine_call(acc1, acc2, asla, v1, v2)
    out_di = _combine_call(acc3, acc4, asla, v3, v4)
    return out_mi, out_di


# speedup vs baseline: 1.1792x; 1.1792x over previous
import jax
import jax.numpy as jnp
from jax.experimental import pallas as pl
from jax.experimental.pallas import tpu as pltpu

N = 4096
D = 512
BM = 256
NB = N // BM


def _feat(vb, wslat_ref, bsla_ref):
    s = jnp.tanh(jnp.dot(vb, wslat_ref[...],
                         preferred_element_type=jnp.float32) + bsla_ref[...])
    return jnp.sum(s, axis=0, keepdims=True)


def _copy(meta_ref, mb_scr, sem, k, slot):
    return pltpu.make_async_copy(
        meta_ref.at[pl.ds(k * BM, BM), :], mb_scr.at[slot], sem.at[slot])


def _pair_body(emb_ref, w1t_ref, w2t_ref, bfc1_ref, bfc2_ref, bias1_ref,
               bias2_ref, p1_ref, p2_ref, wslat_ref, bsla_ref, asla_ref,
               meta1_ref, meta2_ref, out_ref,
               mb1_scr, mb2_scr, h1_scr, h2_scr, v1_scr, v2_scr,
               acc1_scr, acc2_scr, sem1, sem2):
    i = pl.program_id(0)

    @pl.when(i == 0)
    def _init():
        _copy(meta1_ref, mb1_scr, sem1, 0, 0).start()
        _copy(meta2_ref, mb2_scr, sem2, 0, 0).start()
        h1 = jnp.dot(emb_ref[...], w1t_ref[...],
                     preferred_element_type=jnp.float32) + bfc1_ref[...]
        h1_scr[...] = h1.astype(jnp.bfloat16)
        h2 = jnp.dot(emb_ref[...], w2t_ref[...],
                     preferred_element_type=jnp.float32) + bfc2_ref[...]
        h2_scr[...] = h2.astype(jnp.bfloat16)
        acc1_scr[...] = jnp.zeros_like(acc1_scr)
        acc2_scr[...] = jnp.zeros_like(acc2_scr)

    @pl.when(i < NB)
    def _heavy():
        @pl.when(i + 1 < NB)
        def _issue_next():
            nxt = i + 1
            nslot = jax.lax.rem(nxt, 2)
            _copy(meta1_ref, mb1_scr, sem1, nxt, nslot).start()
            _copy(meta2_ref, mb2_scr, sem2, nxt, nslot).start()

        slot = jax.lax.rem(i, 2)
        _copy(meta1_ref, mb1_scr, sem1, i, slot).wait()
        _copy(meta2_ref, mb2_scr, sem2, i, slot).wait()

        m1 = mb1_scr[slot].astype(jnp.bfloat16)
        out1 = jnp.dot(m1, h1_scr[...],
                       preferred_element_type=jnp.float32) + bias1_ref[...]
        vb1 = jnp.where(out1 >= 0, out1,
                        p1_ref[0, 0] * out1).astype(jnp.bfloat16)
        v1_scr[pl.ds(i * BM, BM), :] = vb1
        m2 = mb2_scr[slot].astype(jnp.bfloat16)
        out2 = jnp.dot(m2, h2_scr[...],
                       preferred_element_type=jnp.float32) + bias2_ref[...]
        vb2 = jnp.where(out2 >= 0, out2,
                        p2_ref[0, 0] * out2).astype(jnp.bfloat16)
        v2_scr[pl.ds(i * BM, BM), :] = vb2

        acc1_scr[...] += _feat(vb1, wslat_ref, bsla_ref)
        acc2_scr[...] += _feat(vb2, wslat_ref, bsla_ref)

    @pl.when(i >= NB)
    def _combine():
        j = i - NB
        la = jnp.sum(asla_ref[...] * acc1_scr[...] * (1.0 / N),
                     axis=1, keepdims=True)
        lb = jnp.sum(asla_ref[...] * acc2_scr[...] * (1.0 / N),
                     axis=1, keepdims=True)
        m = jnp.maximum(la, lb)
        ea = jnp.exp(la - m)
        eb = jnp.exp(lb - m)
        inv = 1.0 / (ea + eb)
        b1 = ea * inv
        b2 = eb * inv
        v1 = v1_scr[pl.ds(j * BM, BM), :].astype(jnp.float32)
        v2 = v2_scr[pl.ds(j * BM, BM), :].astype(jnp.float32)
        out_ref[...] = v1 * b1 + v2 * b2


def _pair_call(emb_bf, w1t, w2t, bfc1, bfc2, bias1, bias2, p1, p2,
               wslat, bsla, asla, meta1, meta2):
    const = lambda i: (0, 0)
    anyspec = pl.BlockSpec(memory_space=pl.ANY)
    return pl.pallas_call(
        _pair_body,
        grid=(2 * NB,),
        in_specs=[
            pl.BlockSpec((N, D), const),
            pl.BlockSpec((D, D), const),
            pl.BlockSpec((D, D), const),
            pl.BlockSpec((1, D), const),
            pl.BlockSpec((1, D), const),
            pl.BlockSpec((1, D), const),
            pl.BlockSpec((1, D), const),
            pl.BlockSpec((1, 1), const),
            pl.BlockSpec((1, 1), const),
            pl.BlockSpec((D, D), const),
            pl.BlockSpec((1, D), const),
            pl.BlockSpec((1, D), const),
            anyspec,
            anyspec,
        ],
        out_specs=pl.BlockSpec((BM, D),
                               lambda i: (jnp.clip(i - NB, 0, NB - 1), 0)),
        out_shape=jax.ShapeDtypeStruct((N, D), jnp.float32),
        scratch_shapes=[
            pltpu.VMEM((2, BM, N), jnp.float32),
            pltpu.VMEM((2, BM, N), jnp.float32),
            pltpu.VMEM((N, D), jnp.bfloat16),
            pltpu.VMEM((N, D), jnp.bfloat16),
            pltpu.VMEM((N, D), jnp.bfloat16),
            pltpu.VMEM((N, D), jnp.bfloat16),
            pltpu.VMEM((1, D), jnp.float32),
            pltpu.VMEM((1, D), jnp.float32),
            pltpu.SemaphoreType.DMA((2,)),
            pltpu.SemaphoreType.DMA((2,)),
        ],
    )(emb_bf, w1t, w2t, bfc1, bfc2, bias1, bias2, p1, p2, wslat, bsla, asla,
      meta1, meta2)


@jax.jit
def kernel(emb_mi, emb_di, meta_mdm, meta_mdmdm, meta_dmd, meta_dmdmd,
           W_mdm, bfc_mdm, bias_mdm, p_mdm,
           W_mdmdm, bfc_mdmdm, bias_mdmdm, p_mdmdm,
           W_dmd, bfc_dmd, bias_dmd, p_dmd,
           W_dmdmd, bfc_dmdmd, bias_dmdmd, p_dmdmd,
           W_sla, b_sla, a_sla):
    wslat = W_sla.T.astype(jnp.bfloat16)
    bsla = b_sla.reshape(1, D)
    asla = a_sla.reshape(1, D)

    out_mi = _pair_call(
        emb_mi.astype(jnp.bfloat16),
        W_mdm.T.astype(jnp.bfloat16), W_mdmdm.T.astype(jnp.bfloat16),
        bfc_mdm.reshape(1, D), bfc_mdmdm.reshape(1, D),
        bias_mdm.reshape(1, D), bias_mdmdm.reshape(1, D),
        p_mdm.reshape(1, 1), p_mdmdm.reshape(1, 1),
        wslat, bsla, asla, meta_mdm, meta_mdmdm)
    out_di = _pair_call(
        emb_di.astype(jnp.bfloat16),
        W_dmd.T.astype(jnp.bfloat16), W_dmdmd.T.astype(jnp.bfloat16),
        bfc_dmd.reshape(1, D), bfc_dmdmd.reshape(1, D),
        bias_dmd.reshape(1, D), bias_dmdmd.reshape(1, D),
        p_dmd.reshape(1, 1), p_dmdmd.reshape(1, 1),
        wslat, bsla, asla, meta_dmd, meta_dmdmd)
    return out_mi, out_di
